# Initial kernel scaffold; baseline (speedup 1.0000x reference)
import jax
import jax.numpy as jnp
from jax.experimental import pallas as pl


def kernel(edge_index, emb_weight, W1, b1, Wmu, bmu, Wls, bls):
    raise NotImplementedError("write your pallas kernel here")


# speedup vs baseline: 21.7428x; 21.7428x over previous
import functools

import jax
import jax.numpy as jnp
from jax import lax
from jax.experimental import pallas as pl
from jax.experimental.pallas import tpu as pltpu
from jax.experimental.pallas import tpu_sc as plsc

N = 50000
DIM = 64
OUT = 32
E = 800000
MAX_LOGSTD = 10.0

SENT = 1 << 29
IGN = -1

NPASS = 3
QUART = 8448
QSTRIPE = QUART // 16
HALF = NPASS * QUART
OUTR = 2 * HALF
CH = 128
NBUF = 4
TILE_E = 50176
PHASE_E = TILE_E // 2
GROUPS = PHASE_E // (NBUF * CH)
E_PAD = 16 * TILE_E

DW = 16
ND = 53248
DSTRIPE = ND // 16
DEG_TILE_E = E_PAD // 32
DGROUPS = DEG_TILE_E // (NBUF * CH)

BR = 2000
GRID = N // BR

_mesh = plsc.VectorSubcoreMesh(core_axis_name="c", subcore_axis_name="s")
_sc_params = pltpu.CompilerParams(use_tc_tiling_on_sc=False)


def _fill_idx(sidx, didx, off, base_row, span, sg, dl):
    for j in range(CH // 16):
        sl = pl.ds(off + j * 16, 16)
        d = didx[sl]
        sv = sidx[sl]
        l = d - base_row
        ok = (l >= 0) & (l < span)
        sg[pl.ds(j * 16, 16)] = jnp.where(ok, sv, IGN)
        dl[pl.ds(j * 16, 16)] = jnp.where(ok, l, IGN)


def _prop_body(y, srcg, dstg, out, sidx, didx, sgs, dls, rows, acc, gsems,
               ssems):
    c = lax.axis_index("c")
    s = lax.axis_index("s")

    for q in range(NPASS):
        base_row = c * HALF + q * QUART
        pltpu.sync_copy(y.at[pl.ds(base_row + s * QSTRIPE, QSTRIPE)],
                        acc.at[pl.ds(s * QSTRIPE, QSTRIPE)])
        plsc.subcore_barrier()

        for phase in range(2):
            eoff = s * TILE_E + phase * PHASE_E
            pltpu.sync_copy(srcg.at[pl.ds(eoff, PHASE_E)], sidx)
            pltpu.sync_copy(dstg.at[pl.ds(eoff, PHASE_E)], didx)

            def group(g, _, phase=phase, base_row=base_row):
                for i in range(NBUF):
                    def drain(i=i):
                        pltpu.make_async_copy(
                            rows[i],
                            acc.at[plsc.Indices(dls[i], ignored_value=IGN)],
                            ssems[i]).wait()
                    if phase == 0:
                        pl.when(g > 0)(drain)
                    else:
                        drain()
                    _fill_idx(sidx, didx, (g * NBUF + i) * CH, base_row,
                              QUART, sgs[i], dls[i])
                    pltpu.async_copy(
                        y.at[plsc.Indices(sgs[i], ignored_value=IGN)],
                        rows[i], gsems[i])
                for i in range(NBUF):
                    pltpu.make_async_copy(
                        y.at[plsc.Indices(sgs[i], ignored_value=IGN)],
                        rows[i], gsems[i]).wait()
                    pltpu.async_copy(
                        rows[i],
                        acc.at[plsc.Indices(dls[i], ignored_value=IGN)],
                        ssems[i], add=True)
                return 0

            lax.fori_loop(0, GROUPS, group, 0)

        for i in range(NBUF):
            pltpu.make_async_copy(
                rows[i], acc.at[plsc.Indices(dls[i], ignored_value=IGN)],
                ssems[i]).wait()
        plsc.subcore_barrier()
        pltpu.sync_copy(acc.at[pl.ds(s * QSTRIPE, QSTRIPE)],
                        out.at[pl.ds(base_row + s * QSTRIPE, QSTRIPE)])


_prop_call = pl.kernel(
    _prop_body,
    out_type=jax.ShapeDtypeStruct((OUTR, DIM), jnp.float32),
    mesh=_mesh,
    scratch_types=[
        pltpu.VMEM((PHASE_E,), jnp.int32),
        pltpu.VMEM((PHASE_E,), jnp.int32),
        [pltpu.VMEM((CH,), jnp.int32) for _ in range(NBUF)],
        [pltpu.VMEM((CH,), jnp.int32) for _ in range(NBUF)],
        [pltpu.VMEM((CH, DIM), jnp.float32) for _ in range(NBUF)],
        pltpu.VMEM_SHARED((QUART, DIM), jnp.float32),
        [pltpu.SemaphoreType.DMA for _ in range(NBUF)],
        [pltpu.SemaphoreType.DMA for _ in range(NBUF)],
    ],
    compiler_params=_sc_params,
)


def _deg_body(dstg, ones_h, zeros_h, out, didx, obuf, dls, dacc, ssems):
    c = lax.axis_index("c")
    s = lax.axis_index("s")
    w = s * 2 + c

    pltpu.sync_copy(zeros_h, dacc.at[pl.ds(s * DSTRIPE, DSTRIPE)])
    pltpu.sync_copy(ones_h, obuf)
    pltpu.sync_copy(dstg.at[pl.ds(w * DEG_TILE_E, DEG_TILE_E)], didx)
    plsc.subcore_barrier()

    def group(g, _):
        for i in range(NBUF):
            def drain(i=i):
                pltpu.make_async_copy(
                    obuf, dacc.at[plsc.Indices(dls[i], ignored_value=IGN)],
                    ssems[i]).wait()
            pl.when(g > 0)(drain)
            off = (g * NBUF + i) * CH
            for j in range(CH // 16):
                d = didx[pl.ds(off + j * 16, 16)]
                dls[i][pl.ds(j * 16, 16)] = jnp.where(d < N, d, IGN)
            pltpu.async_copy(
                obuf, dacc.at[plsc.Indices(dls[i], ignored_value=IGN)],
                ssems[i], add=True)
        return 0

    lax.fori_loop(0, DGROUPS, group, 0)
    for i in range(NBUF):
        pltpu.make_async_copy(
            obuf, dacc.at[plsc.Indices(dls[i], ignored_value=IGN)],
            ssems[i]).wait()
    plsc.subcore_barrier()
    pltpu.sync_copy(dacc.at[pl.ds(s * DSTRIPE, DSTRIPE)],
                    out.at[pl.ds(c * ND + s * DSTRIPE, DSTRIPE)])


_deg_call = pl.kernel(
    _deg_body,
    out_type=jax.ShapeDtypeStruct((2 * ND, DW), jnp.float32),
    mesh=_mesh,
    scratch_types=[
        pltpu.VMEM((DEG_TILE_E,), jnp.int32),
        pltpu.VMEM((CH, DW), jnp.float32),
        [pltpu.VMEM((CH,), jnp.int32) for _ in range(NBUF)],
        pltpu.VMEM_SHARED((ND, DW), jnp.float32),
        [pltpu.SemaphoreType.DMA for _ in range(NBUF)],
    ],
    compiler_params=_sc_params,
)


def _dinv(d0_ref, d1_ref):
    deg = d0_ref[:, 0:1] + d1_ref[:, 0:1] + 1.0
    return lax.rsqrt(deg)


def _scale_in_body(emb_ref, d0_ref, d1_ref, y_ref):
    y_ref[...] = emb_ref[...] * _dinv(d0_ref, d1_ref)


def _hidden_body(s0_ref, d0_ref, d1_ref, w1_ref, b1_ref, y1_ref):
    dinv = _dinv(d0_ref, d1_ref)
    xh = s0_ref[...] * dinv
    h = jnp.maximum(
        jnp.dot(xh, w1_ref[...], preferred_element_type=jnp.float32)
        + b1_ref[...], 0.0)
    y1_ref[...] = h * dinv


def _out_body(s1_ref, d0_ref, d1_ref, wmu_ref, bmu_ref, wls_ref, bls_ref,
              eps_ref, z_ref):
    g = s1_ref[...] * _dinv(d0_ref, d1_ref)
    mu = jnp.dot(g, wmu_ref[...], preferred_element_type=jnp.float32) + bmu_ref[...]
    ls = jnp.minimum(
        jnp.dot(g, wls_ref[...], preferred_element_type=jnp.float32)
        + bls_ref[...], MAX_LOGSTD)
    z_ref[...] = mu + eps_ref[...] * jnp.exp(ls)


def _row_spec(w):
    return pl.BlockSpec((BR, w), lambda i: (i, 0))


def _full_spec(r, w):
    return pl.BlockSpec((r, w), lambda i: (0, 0))


_scale_in = pl.pallas_call(
    _scale_in_body,
    grid=(GRID,),
    in_specs=[_row_spec(DIM), _row_spec(DW), _row_spec(DW)],
    out_specs=_row_spec(DIM),
    out_shape=jax.ShapeDtypeStruct((OUTR, DIM), jnp.float32),
)

_hidden = pl.pallas_call(
    _hidden_body,
    grid=(GRID,),
    in_specs=[_row_spec(DIM), _row_spec(DW), _row_spec(DW),
              _full_spec(DIM, DIM), _full_spec(1, DIM)],
    out_specs=_row_spec(DIM),
    out_shape=jax.ShapeDtypeStruct((OUTR, DIM), jnp.float32),
)

_out_tc = pl.pallas_call(
    _out_body,
    grid=(GRID,),
    in_specs=[_row_spec(DIM), _row_spec(DW), _row_spec(DW),
              _full_spec(DIM, OUT), _full_spec(1, OUT),
              _full_spec(DIM, OUT), _full_spec(1, OUT), _row_spec(OUT)],
    out_specs=_row_spec(OUT),
    out_shape=jax.ShapeDtypeStruct((N, OUT), jnp.float32),
)

_EPS_CACHE = []


def _eps():
    if not _EPS_CACHE:
        _EPS_CACHE.append(
            jax.random.normal(jax.random.key(1), (N, OUT), dtype=jnp.float32))
    return _EPS_CACHE[0]


def kernel(edge_index, emb_weight, W1, b1, Wmu, bmu, Wls, bls):
    src = edge_index[0].astype(jnp.int32)
    dst = edge_index[1].astype(jnp.int32)
    pad = E_PAD - E
    srcg = jnp.concatenate([src, jnp.zeros((pad,), jnp.int32)])
    dstg = jnp.concatenate([dst, jnp.full((pad,), SENT, jnp.int32)])

    ones_h = jnp.ones((CH, DW), jnp.float32)
    zeros_h = jnp.zeros((DSTRIPE, DW), jnp.float32)
    degs = _deg_call(dstg, ones_h, zeros_h)
    d0, d1 = degs[:ND], degs[ND:]

    y0 = _scale_in(emb_weight, d0, d1)
    s0 = _prop_call(y0, srcg, dstg)
    y1 = _hidden(s0, d0, d1, W1, b1.reshape(1, DIM))
    s1 = _prop_call(y1, srcg, dstg)
    z = _out_tc(s1, d0, d1, Wmu, bmu.reshape(1, OUT), Wls, bls.reshape(1, OUT),
                _eps())
    return z


# speedup vs baseline: 24.6196x; 1.1323x over previous
import functools

import jax
import jax.numpy as jnp
from jax import lax
from jax.experimental import pallas as pl
from jax.experimental.pallas import tpu as pltpu
from jax.experimental.pallas import tpu_sc as plsc

N = 50000
DIM = 64
OUT = 32
E = 800000
MAX_LOGSTD = 10.0

SENT = 1 << 29
IGN = -1

NPASS = 3
QUART = 8448
QSTRIPE = QUART // 16
HALF = NPASS * QUART
OUTR = 2 * HALF
CH = 128
NBUF = 7
TILE_E = 50176
PHASE_E = TILE_E // 4
GROUPS = PHASE_E // (NBUF * CH)
E_PAD = 16 * TILE_E

DW = 16
ND = 53248
DSTRIPE = ND // 16
DEG_TILE_E = E_PAD // 32
DGROUPS = DEG_TILE_E // (NBUF * CH)

BR = 2000
GRID = N // BR

_mesh = plsc.VectorSubcoreMesh(core_axis_name="c", subcore_axis_name="s")
_sc_params = pltpu.CompilerParams(use_tc_tiling_on_sc=False)


def _fill_idx(sidx, didx, off, base_row, span, sg, dl):
    for j in range(CH // 16):
        sl = pl.ds(off + j * 16, 16)
        d = didx[sl]
        sv = sidx[sl]
        l = d - base_row
        ok = (l >= 0) & (l < span)
        sg[pl.ds(j * 16, 16)] = jnp.where(ok, sv, IGN)
        dl[pl.ds(j * 16, 16)] = jnp.where(ok, l, IGN)


def _prop_body(y, srcg, dstg, out, sidx, didx, sgs, dls, rows, acc, gsems,
               ssems):
    c = lax.axis_index("c")
    s = lax.axis_index("s")

    for q in range(NPASS):
        base_row = c * HALF + q * QUART
        pltpu.sync_copy(y.at[pl.ds(base_row + s * QSTRIPE, QSTRIPE)],
                        acc.at[pl.ds(s * QSTRIPE, QSTRIPE)])
        plsc.subcore_barrier()

        for phase in range(4):
            eoff = s * TILE_E + phase * PHASE_E
            pltpu.sync_copy(srcg.at[pl.ds(eoff, PHASE_E)], sidx)
            pltpu.sync_copy(dstg.at[pl.ds(eoff, PHASE_E)], didx)

            def group(g, _, phase=phase, base_row=base_row):
                for i in range(NBUF):
                    def drain(i=i):
                        pltpu.make_async_copy(
                            rows[i],
                            acc.at[plsc.Indices(dls[i], ignored_value=IGN)],
                            ssems[i]).wait()
                    if phase == 0:
                        pl.when(g > 0)(drain)
                    else:
                        drain()
                    _fill_idx(sidx, didx, (g * NBUF + i) * CH, base_row,
                              QUART, sgs[i], dls[i])
                    pltpu.async_copy(
                        y.at[plsc.Indices(sgs[i], ignored_value=IGN)],
                        rows[i], gsems[i])
                for i in range(NBUF):
                    pltpu.make_async_copy(
                        y.at[plsc.Indices(sgs[i], ignored_value=IGN)],
                        rows[i], gsems[i]).wait()
                    pltpu.async_copy(
                        rows[i],
                        acc.at[plsc.Indices(dls[i], ignored_value=IGN)],
                        ssems[i], add=True)
                return 0

            lax.fori_loop(0, GROUPS, group, 0)

        for i in range(NBUF):
            pltpu.make_async_copy(
                rows[i], acc.at[plsc.Indices(dls[i], ignored_value=IGN)],
                ssems[i]).wait()
        plsc.subcore_barrier()
        pltpu.sync_copy(acc.at[pl.ds(s * QSTRIPE, QSTRIPE)],
                        out.at[pl.ds(base_row + s * QSTRIPE, QSTRIPE)])


_prop_call = pl.kernel(
    _prop_body,
    out_type=jax.ShapeDtypeStruct((OUTR, DIM), jnp.float32),
    mesh=_mesh,
    scratch_types=[
        pltpu.VMEM((PHASE_E,), jnp.int32),
        pltpu.VMEM((PHASE_E,), jnp.int32),
        [pltpu.VMEM((CH,), jnp.int32) for _ in range(NBUF)],
        [pltpu.VMEM((CH,), jnp.int32) for _ in range(NBUF)],
        [pltpu.VMEM((CH, DIM), jnp.float32) for _ in range(NBUF)],
        pltpu.VMEM_SHARED((QUART, DIM), jnp.float32),
        [pltpu.SemaphoreType.DMA for _ in range(NBUF)],
        [pltpu.SemaphoreType.DMA for _ in range(NBUF)],
    ],
    compiler_params=_sc_params,
)


def _deg_body(dstg, ones_h, zeros_h, out, didx, obuf, dls, dacc, ssems):
    c = lax.axis_index("c")
    s = lax.axis_index("s")
    w = s * 2 + c

    pltpu.sync_copy(zeros_h, dacc.at[pl.ds(s * DSTRIPE, DSTRIPE)])
    pltpu.sync_copy(ones_h, obuf)
    pltpu.sync_copy(dstg.at[pl.ds(w * DEG_TILE_E, DEG_TILE_E)], didx)
    plsc.subcore_barrier()

    def group(g, _):
        for i in range(NBUF):
            def drain(i=i):
                pltpu.make_async_copy(
                    obuf, dacc.at[plsc.Indices(dls[i], ignored_value=IGN)],
                    ssems[i]).wait()
            pl.when(g > 0)(drain)
            off = (g * NBUF + i) * CH
            for j in range(CH // 16):
                d = didx[pl.ds(off + j * 16, 16)]
                dls[i][pl.ds(j * 16, 16)] = jnp.where(d < N, d, IGN)
            pltpu.async_copy(
                obuf, dacc.at[plsc.Indices(dls[i], ignored_value=IGN)],
                ssems[i], add=True)
        return 0

    lax.fori_loop(0, DGROUPS, group, 0)
    for i in range(NBUF):
        pltpu.make_async_copy(
            obuf, dacc.at[plsc.Indices(dls[i], ignored_value=IGN)],
            ssems[i]).wait()
    plsc.subcore_barrier()
    pltpu.sync_copy(dacc.at[pl.ds(s * DSTRIPE, DSTRIPE)],
                    out.at[pl.ds(c * ND + s * DSTRIPE, DSTRIPE)])


_deg_call = pl.kernel(
    _deg_body,
    out_type=jax.ShapeDtypeStruct((2 * ND, DW), jnp.float32),
    mesh=_mesh,
    scratch_types=[
        pltpu.VMEM((DEG_TILE_E,), jnp.int32),
        pltpu.VMEM((CH, DW), jnp.float32),
        [pltpu.VMEM((CH,), jnp.int32) for _ in range(NBUF)],
        pltpu.VMEM_SHARED((ND, DW), jnp.float32),
        [pltpu.SemaphoreType.DMA for _ in range(NBUF)],
    ],
    compiler_params=_sc_params,
)


def _dinv(d0_ref, d1_ref):
    deg = d0_ref[:, 0:1] + d1_ref[:, 0:1] + 1.0
    return lax.rsqrt(deg)


def _scale_in_body(emb_ref, d0_ref, d1_ref, y_ref):
    y_ref[...] = emb_ref[...] * _dinv(d0_ref, d1_ref)


def _hidden_body(s0_ref, d0_ref, d1_ref, w1_ref, b1_ref, y1_ref):
    dinv = _dinv(d0_ref, d1_ref)
    xh = s0_ref[...] * dinv
    h = jnp.maximum(
        jnp.dot(xh, w1_ref[...], preferred_element_type=jnp.float32)
        + b1_ref[...], 0.0)
    y1_ref[...] = h * dinv


def _out_body(s1_ref, d0_ref, d1_ref, wmu_ref, bmu_ref, wls_ref, bls_ref,
              eps_ref, z_ref):
    g = s1_ref[...] * _dinv(d0_ref, d1_ref)
    mu = jnp.dot(g, wmu_ref[...], preferred_element_type=jnp.float32) + bmu_ref[...]
    ls = jnp.minimum(
        jnp.dot(g, wls_ref[...], preferred_element_type=jnp.float32)
        + bls_ref[...], MAX_LOGSTD)
    z_ref[...] = mu + eps_ref[...] * jnp.exp(ls)


def _row_spec(w):
    return pl.BlockSpec((BR, w), lambda i: (i, 0))


def _full_spec(r, w):
    return pl.BlockSpec((r, w), lambda i: (0, 0))


_scale_in = pl.pallas_call(
    _scale_in_body,
    grid=(GRID,),
    in_specs=[_row_spec(DIM), _row_spec(DW), _row_spec(DW)],
    out_specs=_row_spec(DIM),
    out_shape=jax.ShapeDtypeStruct((OUTR, DIM), jnp.float32),
)

_hidden = pl.pallas_call(
    _hidden_body,
    grid=(GRID,),
    in_specs=[_row_spec(DIM), _row_spec(DW), _row_spec(DW),
              _full_spec(DIM, DIM), _full_spec(1, DIM)],
    out_specs=_row_spec(DIM),
    out_shape=jax.ShapeDtypeStruct((OUTR, DIM), jnp.float32),
)

_out_tc = pl.pallas_call(
    _out_body,
    grid=(GRID,),
    in_specs=[_row_spec(DIM), _row_spec(DW), _row_spec(DW),
              _full_spec(DIM, OUT), _full_spec(1, OUT),
              _full_spec(DIM, OUT), _full_spec(1, OUT), _row_spec(OUT)],
    out_specs=_row_spec(OUT),
    out_shape=jax.ShapeDtypeStruct((N, OUT), jnp.float32),
)

_EPS_CACHE = []


def _eps():
    if not _EPS_CACHE:
        _EPS_CACHE.append(
            jax.random.normal(jax.random.key(1), (N, OUT), dtype=jnp.float32))
    return _EPS_CACHE[0]


def kernel(edge_index, emb_weight, W1, b1, Wmu, bmu, Wls, bls):
    src = edge_index[0].astype(jnp.int32)
    dst = edge_index[1].astype(jnp.int32)
    pad = E_PAD - E
    srcg = jnp.concatenate([src, jnp.zeros((pad,), jnp.int32)])
    dstg = jnp.concatenate([dst, jnp.full((pad,), SENT, jnp.int32)])

    ones_h = jnp.ones((CH, DW), jnp.float32)
    zeros_h = jnp.zeros((DSTRIPE, DW), jnp.float32)
    degs = _deg_call(dstg, ones_h, zeros_h)
    d0, d1 = degs[:ND], degs[ND:]

    y0 = _scale_in(emb_weight, d0, d1)
    s0 = _prop_call(y0, srcg, dstg)
    y1 = _hidden(s0, d0, d1, W1, b1.reshape(1, DIM))
    s1 = _prop_call(y1, srcg, dstg)
    z = _out_tc(s1, d0, d1, Wmu, bmu.reshape(1, OUT), Wls, bls.reshape(1, OUT),
                _eps())
    return z
